# baseline (speedup 1.0000x reference)
import jax
import jax.numpy as jnp
from jax.experimental import pallas as pl

_B, _C_IN, _L = 256, 64, 512
_C_OUT, _KW = 128, 8
_LOUT = _L - _KW + 1
_BN = 8


def _fused_encoder_kernel(x_ref, wt_ref, b_ref, o_ref):
    acc = jnp.zeros((_BN, _LOUT, _C_OUT), jnp.float32)
    for k in range(_KW):
        xk = x_ref[:, :, k:k + _LOUT]
        wk = wt_ref[k]
        acc = acc + jax.lax.dot_general(
            xk, wk, (((1,), (0,)), ((), ())),
            preferred_element_type=jnp.float32)
    acc = acc + b_ref[0][None, None, :]
    acc = jnp.maximum(acc, 0.0)
    q = jnp.sum(acc, axis=1) * (1.0 / _LOUT)
    nrm = jnp.sqrt(jnp.sum(q * q, axis=1, keepdims=True))
    o_ref[...] = q / jnp.maximum(nrm, 1e-12)


@jax.jit
def kernel(sequence_q_s, W, b):
    wt = W.transpose(2, 1, 0)
    b2 = b.reshape(1, _C_OUT)
    return pl.pallas_call(
        _fused_encoder_kernel,
        grid=(_B // _BN,),
        in_specs=[
            pl.BlockSpec((_BN, _C_IN, _L), lambda i: (i, 0, 0)),
            pl.BlockSpec((_KW, _C_IN, _C_OUT), lambda i: (0, 0, 0)),
            pl.BlockSpec((1, _C_OUT), lambda i: (0, 0)),
        ],
        out_specs=pl.BlockSpec((_BN, _C_OUT), lambda i: (i, 0)),
        out_shape=jax.ShapeDtypeStruct((_B, _C_OUT), jnp.float32),
    )(sequence_q_s, wt, b2)


# speedup vs baseline: 2.5599x; 2.5599x over previous
import jax
import jax.numpy as jnp
from jax.experimental import pallas as pl

_B, _C_IN, _L = 256, 64, 512
_C_OUT, _KW = 128, 8
_LOUT = _L - _KW + 1
_BN = 8


def _fused_encoder_kernel(x_ref, w_ref, b_ref, o_ref):
    x = x_ref[...].astype(jnp.bfloat16)
    acc = jnp.zeros((_C_OUT, _BN, _LOUT), jnp.float32)
    for k in range(_KW):
        xk = x[:, :, k:k + _LOUT]
        wk = w_ref[k].astype(jnp.bfloat16)
        acc = acc + jax.lax.dot_general(
            wk, xk, (((1,), (1,)), ((), ())),
            preferred_element_type=jnp.float32)
    acc = acc + b_ref[...][:, :, None]
    acc = jnp.maximum(acc, 0.0)
    q = jnp.sum(acc, axis=2) * (1.0 / _LOUT)
    nrm = jnp.sqrt(jnp.sum(q * q, axis=0, keepdims=True))
    q = q / jnp.maximum(nrm, 1e-12)
    o_ref[...] = q.T


@jax.jit
def kernel(sequence_q_s, W, b):
    wt = W.transpose(2, 0, 1)
    b2 = b.reshape(_C_OUT, 1)
    return pl.pallas_call(
        _fused_encoder_kernel,
        grid=(_B // _BN,),
        in_specs=[
            pl.BlockSpec((_BN, _C_IN, _L), lambda i: (i, 0, 0)),
            pl.BlockSpec((_KW, _C_OUT, _C_IN), lambda i: (0, 0, 0)),
            pl.BlockSpec((_C_OUT, 1), lambda i: (0, 0)),
        ],
        out_specs=pl.BlockSpec((_BN, _C_OUT), lambda i: (i, 0)),
        out_shape=jax.ShapeDtypeStruct((_B, _C_OUT), jnp.float32),
    )(sequence_q_s, wt, b2)


# speedup vs baseline: 15.0947x; 5.8965x over previous
import jax
import jax.numpy as jnp
from jax.experimental import pallas as pl

_B, _C_IN, _L = 256, 64, 512
_C_OUT, _KW = 128, 8
_LOUT = _L - _KW + 1
_BN = 8


def _fused_encoder_kernel(x_ref, w_ref, b_ref, o_ref):
    xb = x_ref[...].astype(jnp.bfloat16)
    rolls = [xb] + [jnp.roll(xb, -k, axis=2) for k in range(1, _KW)]
    im = jnp.concatenate(rolls, axis=1)
    w = w_ref[...].astype(jnp.bfloat16)
    bias = b_ref[...]
    tmask = (jax.lax.broadcasted_iota(jnp.int32, (1, _L), 1) < _LOUT)
    cols = []
    for bi in range(_BN):
        y = jax.lax.dot_general(
            w, im[bi], (((1,), (0,)), ((), ())),
            preferred_element_type=jnp.float32)
        y = jnp.maximum(y + bias, 0.0)
        y = jnp.where(tmask, y, 0.0)
        cols.append(jnp.sum(y, axis=1, keepdims=True))
    q = jnp.concatenate(cols, axis=1) * (1.0 / _LOUT)
    nrm = jnp.sqrt(jnp.sum(q * q, axis=0, keepdims=True))
    q = q / jnp.maximum(nrm, 1e-12)
    o_ref[...] = q.T


@jax.jit
def kernel(sequence_q_s, W, b):
    wf = W.transpose(0, 2, 1).reshape(_C_OUT, _KW * _C_IN)
    b2 = b.reshape(_C_OUT, 1)
    return pl.pallas_call(
        _fused_encoder_kernel,
        grid=(_B // _BN,),
        in_specs=[
            pl.BlockSpec((_BN, _C_IN, _L), lambda i: (i, 0, 0)),
            pl.BlockSpec((_C_OUT, _KW * _C_IN), lambda i: (0, 0)),
            pl.BlockSpec((_C_OUT, 1), lambda i: (0, 0)),
        ],
        out_specs=pl.BlockSpec((_BN, _C_OUT), lambda i: (i, 0)),
        out_shape=jax.ShapeDtypeStruct((_B, _C_OUT), jnp.float32),
    )(sequence_q_s, wf, b2)


# speedup vs baseline: 16.9261x; 1.1213x over previous
import jax
import jax.numpy as jnp
from jax.experimental import pallas as pl

_B, _C_IN, _L = 256, 64, 512
_C_OUT, _KW = 128, 8
_LOUT = _L - _KW + 1
_BN = 32


def _fused_encoder_kernel(x_ref, w_ref, b_ref, o_ref):
    xb = x_ref[...].astype(jnp.bfloat16)
    rolls = [xb] + [jnp.roll(xb, -k, axis=2) for k in range(1, _KW)]
    im = jnp.concatenate(rolls, axis=1)
    w = w_ref[...].astype(jnp.bfloat16)
    bias = b_ref[...]
    tmask = (jax.lax.broadcasted_iota(jnp.int32, (1, _L), 1) < _LOUT)
    cols = []
    for bi in range(_BN):
        y = jax.lax.dot_general(
            w, im[bi], (((1,), (0,)), ((), ())),
            preferred_element_type=jnp.float32)
        y = jnp.maximum(y + bias, 0.0)
        y = jnp.where(tmask, y, 0.0)
        cols.append(jnp.sum(y, axis=1, keepdims=True))
    q = jnp.concatenate(cols, axis=1) * (1.0 / _LOUT)
    nrm = jnp.sqrt(jnp.sum(q * q, axis=0, keepdims=True))
    q = q / jnp.maximum(nrm, 1e-12)
    o_ref[...] = q.T


@jax.jit
def kernel(sequence_q_s, W, b):
    wf = W.transpose(0, 2, 1).reshape(_C_OUT, _KW * _C_IN)
    b2 = b.reshape(_C_OUT, 1)
    return pl.pallas_call(
        _fused_encoder_kernel,
        grid=(_B // _BN,),
        in_specs=[
            pl.BlockSpec((_BN, _C_IN, _L), lambda i: (i, 0, 0)),
            pl.BlockSpec((_C_OUT, _KW * _C_IN), lambda i: (0, 0)),
            pl.BlockSpec((_C_OUT, 1), lambda i: (0, 0)),
        ],
        out_specs=pl.BlockSpec((_BN, _C_OUT), lambda i: (i, 0)),
        out_shape=jax.ShapeDtypeStruct((_B, _C_OUT), jnp.float32),
    )(sequence_q_s, wf, b2)


# speedup vs baseline: 17.0254x; 1.0059x over previous
import jax
import jax.numpy as jnp
from jax.experimental import pallas as pl
from jax.experimental.pallas import tpu as pltpu

_B, _C_IN, _L = 256, 64, 512
_C_OUT, _KW = 128, 8
_LOUT = _L - _KW + 1
_BN = 32


def _fused_encoder_kernel(x_ref, w_ref, b_ref, o_ref):
    xb = x_ref[...].astype(jnp.bfloat16)
    rolls = [xb] + [jnp.roll(xb, -k, axis=2) for k in range(1, _KW)]
    im = jnp.concatenate(rolls, axis=1)
    w = w_ref[...].astype(jnp.bfloat16)
    bias = b_ref[...]
    tmask = (jax.lax.broadcasted_iota(jnp.int32, (1, _L), 1) < _LOUT)
    cols = []
    for bi in range(_BN):
        y = jax.lax.dot_general(
            w, im[bi], (((1,), (0,)), ((), ())),
            preferred_element_type=jnp.float32)
        y = jnp.maximum(y + bias, 0.0)
        y = jnp.where(tmask, y, 0.0)
        cols.append(jnp.sum(y, axis=1, keepdims=True))
    q = jnp.concatenate(cols, axis=1) * (1.0 / _LOUT)
    nrm = jnp.sqrt(jnp.sum(q * q, axis=0, keepdims=True))
    q = q / jnp.maximum(nrm, 1e-12)
    o_ref[...] = q.T


@jax.jit
def kernel(sequence_q_s, W, b):
    wf = W.transpose(0, 2, 1).reshape(_C_OUT, _KW * _C_IN)
    b2 = b.reshape(_C_OUT, 1)
    return pl.pallas_call(
        _fused_encoder_kernel,
        grid=(_B // _BN,),
        in_specs=[
            pl.BlockSpec((_BN, _C_IN, _L), lambda i: (i, 0, 0)),
            pl.BlockSpec((_C_OUT, _KW * _C_IN), lambda i: (0, 0)),
            pl.BlockSpec((_C_OUT, 1), lambda i: (0, 0)),
        ],
        out_specs=pl.BlockSpec((_BN, _C_OUT), lambda i: (i, 0)),
        out_shape=jax.ShapeDtypeStruct((_B, _C_OUT), jnp.float32),
        compiler_params=pltpu.CompilerParams(
            dimension_semantics=("parallel",)),
    )(sequence_q_s, wf, b2)


# speedup vs baseline: 17.7844x; 1.0446x over previous
import jax
import jax.numpy as jnp
from jax.experimental import pallas as pl
from jax.experimental.pallas import tpu as pltpu

_B, _C_IN, _L = 256, 64, 512
_C_OUT, _KW = 128, 8
_LOUT = _L - _KW + 1
_K = _KW * _C_IN
_KP = _K + 8
_BN = 32


def _shift0(x, k):
    return jnp.pad(x[:, :, k:k + _LOUT], ((0, 0), (0, 0), (0, _L - _LOUT)))


def _fused_encoder_kernel(x_ref, w_ref, mv_ref, o_ref):
    xb = x_ref[...].astype(jnp.bfloat16)
    parts = [_shift0(xb, k) for k in range(_KW)]
    mrow = jnp.broadcast_to(mv_ref[...][None], (_BN, 8, _L))
    parts.append(mrow)
    im = jnp.concatenate(parts, axis=1)
    w = w_ref[...].astype(jnp.bfloat16)
    cols = []
    for bi in range(_BN):
        y = jax.lax.dot_general(
            w, im[bi], (((1,), (0,)), ((), ())),
            preferred_element_type=jnp.float32)
        y = jnp.maximum(y, 0.0)
        cols.append(jnp.sum(y, axis=1, keepdims=True))
    q = jnp.concatenate(cols, axis=1) * (1.0 / _LOUT)
    nrm = jnp.sqrt(jnp.sum(q * q, axis=0, keepdims=True))
    q = q / jnp.maximum(nrm, 1e-12)
    o_ref[...] = q.T


@jax.jit
def kernel(sequence_q_s, W, b):
    wf = W.transpose(0, 2, 1).reshape(_C_OUT, _K)
    wf = jnp.concatenate(
        [wf, b.reshape(_C_OUT, 1),
         jnp.zeros((_C_OUT, 7), jnp.float32)], axis=1)
    mv = (jnp.arange(_L) < _LOUT).astype(jnp.bfloat16).reshape(1, _L)
    return pl.pallas_call(
        _fused_encoder_kernel,
        grid=(_B // _BN,),
        in_specs=[
            pl.BlockSpec((_BN, _C_IN, _L), lambda i: (i, 0, 0)),
            pl.BlockSpec((_C_OUT, _KP), lambda i: (0, 0)),
            pl.BlockSpec((1, _L), lambda i: (0, 0)),
        ],
        out_specs=pl.BlockSpec((_BN, _C_OUT), lambda i: (i, 0)),
        out_shape=jax.ShapeDtypeStruct((_B, _C_OUT), jnp.float32),
        compiler_params=pltpu.CompilerParams(
            dimension_semantics=("parallel",)),
    )(sequence_q_s, wf, mv)


# speedup vs baseline: 17.8608x; 1.0043x over previous
import jax
import jax.numpy as jnp
from jax.experimental import pallas as pl
from jax.experimental.pallas import tpu as pltpu

_B, _C_IN, _L = 256, 64, 512
_C_OUT, _KW = 128, 8
_LOUT = _L - _KW + 1
_K = _KW * _C_IN
_KP = _K + 8
_BN = 64


def _shift0(x, k):
    return jnp.pad(x[:, :, k:k + _LOUT], ((0, 0), (0, 0), (0, _L - _LOUT)))


def _fused_encoder_kernel(x_ref, w_ref, mv_ref, o_ref):
    xb = x_ref[...].astype(jnp.bfloat16)
    parts = [_shift0(xb, k) for k in range(_KW)]
    mrow = jnp.broadcast_to(mv_ref[...][None], (_BN, 8, _L))
    parts.append(mrow)
    im = jnp.concatenate(parts, axis=1)
    w = w_ref[...].astype(jnp.bfloat16)
    cols = []
    for bi in range(_BN):
        y = jax.lax.dot_general(
            w, im[bi], (((1,), (0,)), ((), ())),
            preferred_element_type=jnp.float32)
        y = jnp.maximum(y, 0.0)
        cols.append(jnp.sum(y, axis=1, keepdims=True))
    q = jnp.concatenate(cols, axis=1) * (1.0 / _LOUT)
    nrm = jnp.sqrt(jnp.sum(q * q, axis=0, keepdims=True))
    q = q / jnp.maximum(nrm, 1e-12)
    o_ref[...] = q.T


@jax.jit
def kernel(sequence_q_s, W, b):
    wf = W.transpose(0, 2, 1).reshape(_C_OUT, _K)
    wf = jnp.concatenate(
        [wf, b.reshape(_C_OUT, 1),
         jnp.zeros((_C_OUT, 7), jnp.float32)], axis=1)
    mv = (jnp.arange(_L) < _LOUT).astype(jnp.bfloat16).reshape(1, _L)
    return pl.pallas_call(
        _fused_encoder_kernel,
        grid=(_B // _BN,),
        in_specs=[
            pl.BlockSpec((_BN, _C_IN, _L), lambda i: (i, 0, 0)),
            pl.BlockSpec((_C_OUT, _KP), lambda i: (0, 0)),
            pl.BlockSpec((1, _L), lambda i: (0, 0)),
        ],
        out_specs=pl.BlockSpec((_BN, _C_OUT), lambda i: (i, 0)),
        out_shape=jax.ShapeDtypeStruct((_B, _C_OUT), jnp.float32),
        compiler_params=pltpu.CompilerParams(
            dimension_semantics=("parallel",)),
    )(sequence_q_s, wf, mv)
